# Initial kernel scaffold; baseline (speedup 1.0000x reference)
import jax
import jax.numpy as jnp
from jax.experimental import pallas as pl


def kernel(x, params):
    raise NotImplementedError("write your pallas kernel here")


# speedup vs baseline: 2.5142x; 2.5142x over previous
import functools

import jax
import jax.numpy as jnp
import numpy as np
from jax.experimental import pallas as pl
from jax.experimental.pallas import tpu as pltpu

D_MODEL = 256
KNN = 16
NBLOCKS = 2
NPOINT = 2048


def _knn_body(src_ref, dst_ref, out_ref, *, n, k):
    s = src_ref[0]
    d = dst_ref[0]
    dn = jnp.sum(d * d, axis=0, keepdims=True)
    cross = jax.lax.dot_general(
        s, d, (((0,), (0,)), ((), ())),
        preferred_element_type=jnp.float32)
    dist = dn - 2.0 * cross
    tm = dist.shape[0]
    iota = jax.lax.broadcasted_iota(jnp.int32, (tm, n), 1)
    kcol = jax.lax.broadcasted_iota(jnp.int32, (tm, k), 1)
    acc = kcol * 0
    for j in range(k):
        m = jnp.min(dist, axis=1, keepdims=True)
        sel = dist <= m
        idx = jnp.min(jnp.where(sel, iota, n), axis=1, keepdims=True)
        acc = acc + (kcol == j).astype(jnp.int32) * idx
        dist = jnp.where(iota == idx, jnp.inf, dist)
    out_ref[0] = acc


def _knn_topk(src_t, dst_t, k):
    b, _, m = src_t.shape
    n = dst_t.shape[2]
    tm = min(m, 256)
    grid = (b, m // tm)
    return pl.pallas_call(
        functools.partial(_knn_body, n=n, k=k),
        grid=grid,
        in_specs=[
            pl.BlockSpec((1, 3, tm), lambda bi, mi: (bi, 0, mi)),
            pl.BlockSpec((1, 3, n), lambda bi, mi: (bi, 0, 0)),
        ],
        out_specs=pl.BlockSpec((1, tm, k), lambda bi, mi: (bi, mi, 0)),
        out_shape=jax.ShapeDtypeStruct((b, m, k), jnp.int32),
    )(src_t, dst_t)


def _fps_body(xyz_ref, out_ref, *, b, n, npoint):
    X = xyz_ref[0]
    Y = xyz_ref[1]
    Z = xyz_ref[2]
    lanes = jax.lax.broadcasted_iota(jnp.int32, (b, n), 1)
    ccol = jax.lax.broadcasted_iota(jnp.int32, (b, npoint), 1)

    def step(i, carry):
        cent, dist, far = carry
        cent = cent + (ccol == i).astype(jnp.int32) * far
        sel = lanes == far
        cx = jnp.sum(jnp.where(sel, X, 0.0), axis=1, keepdims=True)
        cy = jnp.sum(jnp.where(sel, Y, 0.0), axis=1, keepdims=True)
        cz = jnp.sum(jnp.where(sel, Z, 0.0), axis=1, keepdims=True)
        dx = X - cx
        dy = Y - cy
        dz = Z - cz
        d = (dx * dx + dy * dy) + dz * dz
        dist = jnp.minimum(dist, d)
        mx = jnp.max(dist, axis=1, keepdims=True)
        far = jnp.min(jnp.where(dist == mx, lanes, n), axis=1, keepdims=True)
        return cent, dist, far

    srow = jax.lax.broadcasted_iota(jnp.int32, (b, npoint), 0)
    cent0 = (ccol + srow) * 0
    dist0 = X * 0.0 + 1e10
    far0 = jnp.min(X * 0, axis=1, keepdims=True).astype(jnp.int32)
    cent, _, _ = jax.lax.fori_loop(0, npoint, step, (cent0, dist0, far0))
    out_ref[...] = cent


def _fps(xyz3, npoint):
    _, b, n = xyz3.shape
    return pl.pallas_call(
        functools.partial(_fps_body, b=b, n=n, npoint=npoint),
        out_shape=jax.ShapeDtypeStruct((b, npoint), jnp.int32),
    )(xyz3)


def _tb_body(xf_ref, fc1w_ref, fc1b_ref, wq_ref, wk_ref, wv_ref,
             fc2w_ref, fc2b_ref, out_ref, q_s, k_s, v_s, *, s_len, tq):
    fc1w = fc1w_ref[...]
    fc1b = fc1b_ref[...]
    wq = wq_ref[...]
    wk = wk_ref[...]
    wv = wv_ref[...]
    nt = s_len // tq
    for st in range(nt):
        xs = xf_ref[0, pl.ds(st * tq, tq), :]
        xm = jnp.dot(xs, fc1w, preferred_element_type=jnp.float32) + fc1b
        q_s[pl.ds(st * tq, tq), :] = jnp.dot(
            xm, wq, preferred_element_type=jnp.float32)
        k_s[pl.ds(st * tq, tq), :] = jnp.dot(
            xm, wk, preferred_element_type=jnp.float32)
        v_s[pl.ds(st * tq, tq), :] = jnp.dot(
            xm, wv, preferred_element_type=jnp.float32)
    fc2w = fc2w_ref[...]
    fc2b = fc2b_ref[...]
    kk = k_s[...]
    vv = v_s[...]
    scale = 1.0 / np.sqrt(256.0)
    for qt in range(nt):
        q = q_s[pl.ds(qt * tq, tq), :]
        logits = jax.lax.dot_general(
            q, kk, (((1,), (1,)), ((), ())),
            preferred_element_type=jnp.float32) * scale
        mx = jnp.max(logits, axis=1, keepdims=True)
        p = jnp.exp(logits - mx)
        sm = jnp.sum(p, axis=1, keepdims=True)
        attn = p / sm
        res = jnp.dot(attn, vv, preferred_element_type=jnp.float32)
        o = jnp.dot(res, fc2w, preferred_element_type=jnp.float32) + fc2b
        o = jax.nn.relu(o)
        xs = xf_ref[0, pl.ds(qt * tq, tq), :]
        out_ref[0, pl.ds(qt * tq, tq), :] = jax.nn.relu(o + xs)


def _transformer_block_long(xf, p):
    b, s_len, dp = xf.shape
    tq = 256
    grid = (b,)
    return pl.pallas_call(
        functools.partial(_tb_body, s_len=s_len, tq=tq),
        grid=grid,
        in_specs=[
            pl.BlockSpec((1, s_len, dp), lambda bi: (bi, 0, 0)),
            pl.BlockSpec((dp, D_MODEL), lambda bi: (0, 0)),
            pl.BlockSpec((D_MODEL,), lambda bi: (0,)),
            pl.BlockSpec((D_MODEL, D_MODEL), lambda bi: (0, 0)),
            pl.BlockSpec((D_MODEL, D_MODEL), lambda bi: (0, 0)),
            pl.BlockSpec((D_MODEL, D_MODEL), lambda bi: (0, 0)),
            pl.BlockSpec((D_MODEL, dp), lambda bi: (0, 0)),
            pl.BlockSpec((dp,), lambda bi: (0,)),
        ],
        out_specs=pl.BlockSpec((1, s_len, dp), lambda bi: (bi, 0, 0)),
        out_shape=jax.ShapeDtypeStruct((b, s_len, dp), jnp.float32),
        scratch_shapes=[
            pltpu.VMEM((s_len, D_MODEL), jnp.float32),
            pltpu.VMEM((s_len, D_MODEL), jnp.float32),
            pltpu.VMEM((s_len, D_MODEL), jnp.float32),
        ],
    )(xf, p['fc1_w'], p['fc1_b'], p['wq'], p['wk'], p['wv'],
      p['fc2_w'], p['fc2_b'])


def _index_points(points, idx):
    return jax.vmap(lambda p, i: p[i])(points, idx)


def _group_transformer(p, feat):
    x = feat @ p['fc1_w'] + p['fc1_b']
    q, k, v = x @ p['wq'], x @ p['wk'], x @ p['wv']
    attn = jnp.matmul(q, jnp.swapaxes(k, -1, -2))
    attn = jax.nn.softmax(attn / np.sqrt(k.shape[-1]), axis=-1)
    res = jnp.matmul(attn, v)
    return jax.nn.relu(res @ p['fc2_w'] + p['fc2_b'])


def _sa_forward(p, npoint, nsample, xyz_t, xyz, points):
    fps_idx = _fps(jnp.transpose(xyz_t, (1, 0, 2)), npoint)
    new_xyz = _index_points(xyz, fps_idx)
    new_xyz_t = jnp.transpose(new_xyz, (0, 2, 1))
    idx = _knn_topk(new_xyz_t, xyz_t, nsample)
    grouped_xyz = _index_points(xyz, idx)
    grouped_norm = grouped_xyz - new_xyz[:, :, None, :]
    grouped_pts = _index_points(points, idx)
    new_points = jnp.concatenate([grouped_norm, grouped_pts], axis=-1)
    new_points = _group_transformer(p['tb'], new_points)
    new_points = jnp.transpose(new_points, (0, 3, 2, 1))
    for c in p['convs']:
        y = jnp.einsum('bcns,oc->bons', new_points, c['w']) \
            + c['b'][None, :, None, None]
        mu = jnp.mean(y, axis=(0, 2, 3), keepdims=True)
        var = jnp.var(y, axis=(0, 2, 3), keepdims=True)
        y = (y - mu) / jnp.sqrt(var + 1e-5)
        y = y * c['bn_g'][None, :, None, None] + c['bn_b'][None, :, None, None]
        new_points = jax.nn.relu(y)
    new_points = jnp.swapaxes(jnp.max(new_points, axis=2), 1, 2)
    return new_xyz, new_xyz_t, new_points


def kernel(x, params):
    xyz = x[..., :3]
    xyz_t = jnp.transpose(xyz, (0, 2, 1))
    knn_idx = _knn_topk(xyz_t, xyz_t, KNN)
    knn_xyz_n = _index_points(x, knn_idx)
    xyz_pos = xyz[:, :, None] - knn_xyz_n[..., :3]
    t = jnp.concatenate((xyz_pos, knn_xyz_n), axis=3)
    t2 = jax.nn.relu(t @ params['fc_delta_w'] + params['fc_delta_b'])
    xf = jnp.max(t2, axis=2)
    xf = jax.nn.relu(xf @ params['linear1_w'] + params['linear1_b'])
    points = jax.nn.relu(_group_transformer(params['t1'], xf) + xf)
    for i in range(NBLOCKS):
        xyz, xyz_t, points = _sa_forward(
            params['sa'][i], NPOINT // 4 ** (i + 1), KNN, xyz_t, xyz, points)
    return points
